# Initial kernel scaffold; baseline (speedup 1.0000x reference)
import jax
import jax.numpy as jnp
from jax.experimental import pallas as pl


def kernel(x, expert_indices, gate_proj, up_proj, down_proj):
    raise NotImplementedError("write your pallas kernel here")


# speedup vs baseline: 4.7217x; 4.7217x over previous
import jax
import jax.numpy as jnp
from jax import lax
from jax.experimental import pallas as pl

_IB = 256


def _ffn_kernel(xp_ref, gate_ref, down_ref, up_ref, mask_ref, out_ref):
    e = pl.program_id(0)
    j = pl.program_id(1)
    xp = xp_ref[...]
    g = lax.dot_general(xp, gate_ref[0], (((1,), (1,)), ((), ())),
                        preferred_element_type=jnp.float32)
    d = lax.dot_general(xp, down_ref[0], (((1,), (1,)), ((), ())),
                        preferred_element_type=jnp.float32)
    h = (g * jax.nn.sigmoid(g)) * d
    p = lax.dot_general(h, up_ref[0], (((1,), (1,)), ((), ())),
                        preferred_element_type=jnp.float32)

    @pl.when(jnp.logical_and(e == 0, j == 0))
    def _():
        out_ref[...] = jnp.zeros_like(out_ref)

    out_ref[...] += p * mask_ref[0]


def kernel(x, expert_indices, gate_proj, up_proj, down_proj):
    T, H = x.shape
    A = expert_indices.shape[1]
    E, I, _ = gate_proj.shape
    P = T * A

    xp = jnp.repeat(x, A, axis=0)
    idx_flat = expert_indices.reshape(-1).astype(jnp.int32)
    onehot = (idx_flat[None, :] ==
              jnp.arange(E, dtype=jnp.int32)[:, None])
    onehot = onehot.astype(jnp.float32)[:, :, None]

    out = pl.pallas_call(
        _ffn_kernel,
        grid=(E, I // _IB),
        in_specs=[
            pl.BlockSpec((P, H), lambda e, j: (0, 0)),
            pl.BlockSpec((1, _IB, H), lambda e, j: (e, j, 0)),
            pl.BlockSpec((1, _IB, H), lambda e, j: (e, j, 0)),
            pl.BlockSpec((1, H, _IB), lambda e, j: (e, 0, j)),
            pl.BlockSpec((1, P, 1), lambda e, j: (e, 0, 0)),
        ],
        out_specs=pl.BlockSpec((P, H), lambda e, j: (0, 0)),
        out_shape=jax.ShapeDtypeStruct((P, H), jnp.float32),
    )(xp, gate_proj, down_proj, up_proj, onehot)
    return out.reshape(T, A, H)


# speedup vs baseline: 6.5599x; 1.3893x over previous
import jax
import jax.numpy as jnp
from jax import lax
from jax.experimental import pallas as pl

_IB = 1408


def _ffn_kernel(xp_ref, gate_ref, down_ref, up_ref, mask_ref, out_ref):
    e = pl.program_id(0)
    j = pl.program_id(1)
    xp = xp_ref[...]
    g = lax.dot_general(xp, gate_ref[0], (((1,), (1,)), ((), ())),
                        preferred_element_type=jnp.float32)
    d = lax.dot_general(xp, down_ref[0], (((1,), (1,)), ((), ())),
                        preferred_element_type=jnp.float32)
    h = (g * jax.nn.sigmoid(g)) * d
    p = lax.dot_general(h, up_ref[0], (((1,), (1,)), ((), ())),
                        preferred_element_type=jnp.float32)

    @pl.when(jnp.logical_and(e == 0, j == 0))
    def _():
        out_ref[...] = jnp.zeros_like(out_ref)

    out_ref[...] += p * mask_ref[0]


def kernel(x, expert_indices, gate_proj, up_proj, down_proj):
    T, H = x.shape
    A = expert_indices.shape[1]
    E, I, _ = gate_proj.shape
    P = T * A

    xp = jnp.repeat(x, A, axis=0)
    idx_flat = expert_indices.reshape(-1).astype(jnp.int32)
    onehot = (idx_flat[None, :] ==
              jnp.arange(E, dtype=jnp.int32)[:, None])
    onehot = onehot.astype(jnp.float32)[:, :, None]

    out = pl.pallas_call(
        _ffn_kernel,
        grid=(E, I // _IB),
        in_specs=[
            pl.BlockSpec((P, H), lambda e, j: (0, 0)),
            pl.BlockSpec((1, _IB, H), lambda e, j: (e, j, 0)),
            pl.BlockSpec((1, _IB, H), lambda e, j: (e, j, 0)),
            pl.BlockSpec((1, H, _IB), lambda e, j: (e, 0, j)),
            pl.BlockSpec((1, P, 1), lambda e, j: (e, 0, 0)),
        ],
        out_specs=pl.BlockSpec((P, H), lambda e, j: (0, 0)),
        out_shape=jax.ShapeDtypeStruct((P, H), jnp.float32),
    )(xp, gate_proj, down_proj, up_proj, onehot)
    return out.reshape(T, A, H)
